# baseline (speedup 1.0000x reference)
import functools

import jax
import jax.numpy as jnp
from jax import lax
from jax.experimental import pallas as pl
from jax.experimental.pallas import tpu as pltpu
from jax.experimental.pallas import tpu_sc as plsc

N_AUTHORS = 100000
DIM = 64
N_SLICES = 6
BATCH = 16384

_info = plsc.get_sparse_core_info()
NC, NS, L = _info.num_cores, _info.num_subcores, _info.num_lanes
NW = NC * NS
P = BATCH // NW
C = 128
NCHUNK = P // C


def _body(feats_hbm, idx_i_hbm, idx_j_hbm, params_hbm, out_hbm,
          par_v, idxi_v, idxj_v, sidxi_v, sidxj_v,
          rows_i_v, rows_j_v, dist_v, fold_v, sem):
    wid = lax.axis_index("s") * NC + lax.axis_index("c")
    base = wid * P

    pltpu.sync_copy(params_hbm, par_v)
    pv = par_v[pl.ds(0, L)]
    pw = par_v[pl.ds(L, L)]
    lanes = lax.iota(jnp.int32, 16)
    e = jnp.exp(pv)
    ep = jnp.exp(pw)
    w = e / (e + ep)
    ws = [w[s] for s in range(N_SLICES)]
    intercept = pv[6]

    zero16 = jnp.zeros((L,), jnp.float32)
    for k in range(L):
        fold_v[k, pl.ds(L, L)] = zero16

    def chunk_body(chunk, _):
        cbase = base + chunk * C
        pltpu.sync_copy(idx_i_hbm.at[pl.ds(cbase, C)], idxi_v)
        pltpu.sync_copy(idx_j_hbm.at[pl.ds(cbase, C)], idxj_v)
        for k in range(C // L):
            sl = pl.ds(k * L, L)
            vi = idxi_v[sl]
            vj = idxj_v[sl]
            for s in range(N_SLICES):
                sidxi_v[s, sl] = vi + (s * N_AUTHORS)
                sidxj_v[s, sl] = vj + (s * N_AUTHORS)
        copies = []
        for s in range(N_SLICES):
            copies.append(pltpu.async_copy(
                feats_hbm.at[sidxi_v.at[s]], rows_i_v.at[s], sem))
            copies.append(pltpu.async_copy(
                feats_hbm.at[sidxj_v.at[s]], rows_j_v.at[s], sem))
        for cp in copies:
            cp.wait()

        def group_body(g, _):
            dvec = jnp.zeros((L,), jnp.float32)
            for k in range(L):
                c = g * L + k
                acc = []
                for d in range(DIM // L):
                    sl = pl.ds(d * L, L)
                    a = (rows_i_v[0, c, sl] - rows_j_v[0, c, sl]) * ws[0]
                    for s in range(1, N_SLICES):
                        a = a + (rows_i_v[s, c, sl]
                                 - rows_j_v[s, c, sl]) * ws[s]
                    acc.append(a)
                sq = acc[0] * acc[0]
                for d in range(1, DIM // L):
                    sq = sq + acc[d] * acc[d]
                x = sq
                for sh in (8, 4, 2, 1):
                    fold_v[k, pl.ds(0, L)] = x
                    x = x + fold_v[k, pl.ds(sh, L)]
                dvec = dvec + jnp.where(lanes == k, x[0], 0.0)
            dist_v[pl.ds(chunk * C + g * L, L)] = dvec
            return 0

        lax.fori_loop(0, C // L, group_body, 0)
        return 0

    lax.fori_loop(0, NCHUNK, chunk_body, 0)

    for k in range(P // L):
        sl = pl.ds(k * L, L)
        d = dist_v[sl]
        dist_v[sl] = 1.0 / (1.0 + jnp.exp(d - intercept))
    pltpu.sync_copy(dist_v, out_hbm.at[pl.ds(base, P)])


@jax.jit
def kernel(idx_i, idx_j, precomputed_features, feature_weights, intercept):
    feats2 = precomputed_features.reshape(N_SLICES * N_AUTHORS, DIM)
    fw = feature_weights.astype(jnp.float32)
    pad = jnp.zeros((16 - N_SLICES - 1,), jnp.float32)
    icpt = intercept.reshape(1).astype(jnp.float32)
    params = jnp.concatenate([
        fw.reshape(-1), icpt, pad,
        fw[::-1].reshape(-1), icpt, pad,
    ])
    mesh = plsc.VectorSubcoreMesh(core_axis_name="c", subcore_axis_name="s")
    fn = functools.partial(
        pl.kernel,
        mesh=mesh,
        compiler_params=pltpu.CompilerParams(use_tc_tiling_on_sc=False),
        out_type=jax.ShapeDtypeStruct((BATCH,), jnp.float32),
        scratch_types=[
            pltpu.VMEM((32,), jnp.float32),
            pltpu.VMEM((C,), jnp.int32),
            pltpu.VMEM((C,), jnp.int32),
            pltpu.VMEM((N_SLICES, C), jnp.int32),
            pltpu.VMEM((N_SLICES, C), jnp.int32),
            pltpu.VMEM((N_SLICES, C, DIM), jnp.float32),
            pltpu.VMEM((N_SLICES, C, DIM), jnp.float32),
            pltpu.VMEM((P,), jnp.float32),
            pltpu.VMEM((L, 2 * L), jnp.float32),
            pltpu.SemaphoreType.DMA,
        ],
    )(_body)
    return fn(feats2, idx_i, idx_j, params)
